# baseline (speedup 1.0000x reference)
import functools

import jax
import jax.numpy as jnp
from jax import lax
from jax.experimental import pallas as pl
from jax.experimental.pallas import tpu as pltpu
from jax.experimental.pallas import tpu_sc as plsc

B = 2048
D = 2048
E = 8
NC = 16
FB = 16
FPE = 2
H = 1024
RH = 1024
K = 2

LOGIT_PAD = 128
TBA = 256
TS = 256
NBLK = (B * K + E * (TS - 1) + TS - 1) // TS
NSLOT = NBLK * TS
NW = 32
EFW = FPE * FB


def _gelu_exact(x):
    return x * 0.5 * (1.0 + jax.lax.erf(x * 0.7071067811865476))


def _router_kernel(hid_ref, feat_ref, g_ref, b_ref, rw1a_ref, rw1b_ref,
                   rb1_ref, rw2_ref, rb2_ref, h_ref, w_ref, i_ref):
    x = hid_ref[...]
    mu = jnp.mean(x, axis=-1, keepdims=True)
    var = jnp.mean((x - mu) ** 2, axis=-1, keepdims=True)
    h = (x - mu) / jnp.sqrt(var + 1e-5) * g_ref[...] + b_ref[...]
    r1 = jnp.dot(h, rw1a_ref[...], preferred_element_type=jnp.float32)
    r1 = r1 + jnp.dot(feat_ref[...], rw1b_ref[...],
                      preferred_element_type=jnp.float32)
    r1 = _gelu_exact(r1 + rb1_ref[...])
    logits = jnp.dot(r1, rw2_ref[...], preferred_element_type=jnp.float32)
    logits = logits + rb2_ref[...]
    iota = jax.lax.broadcasted_iota(jnp.int32, logits.shape, 1)
    v1 = jnp.max(logits, axis=-1, keepdims=True)
    i1 = jnp.min(jnp.where(logits == v1, iota, LOGIT_PAD), axis=-1,
                 keepdims=True)
    masked = jnp.where(iota == i1, -jnp.inf, logits)
    v2 = jnp.max(masked, axis=-1, keepdims=True)
    i2 = jnp.min(jnp.where(masked == v2, iota, LOGIT_PAD), axis=-1,
                 keepdims=True)
    w1 = jax.nn.sigmoid(v1 - v2)
    w2 = 1.0 - w1
    w = jnp.where(iota == i1, w1, 0.0) + jnp.where(iota == i2, w2, 0.0)
    h_ref[...] = h.astype(jnp.bfloat16)
    w_ref[...] = w
    i_ref[...] = jnp.where(iota == 0, i1, jnp.where(iota == 1, i2, 0))


def _run_router(hidden, feats, ln_gamma, ln_beta, rW1, rb1, rW2p, rb2p):
    return pl.pallas_call(
        _router_kernel,
        grid=(B // TBA,),
        in_specs=[
            pl.BlockSpec((TBA, D), lambda i: (i, 0)),
            pl.BlockSpec((TBA, NC * FB), lambda i: (i, 0)),
            pl.BlockSpec((1, D), lambda i: (0, 0)),
            pl.BlockSpec((1, D), lambda i: (0, 0)),
            pl.BlockSpec((D, RH), lambda i: (0, 0)),
            pl.BlockSpec((NC * FB, RH), lambda i: (0, 0)),
            pl.BlockSpec((1, RH), lambda i: (0, 0)),
            pl.BlockSpec((RH, LOGIT_PAD), lambda i: (0, 0)),
            pl.BlockSpec((1, LOGIT_PAD), lambda i: (0, 0)),
        ],
        out_specs=[
            pl.BlockSpec((TBA, D), lambda i: (i, 0)),
            pl.BlockSpec((TBA, LOGIT_PAD), lambda i: (i, 0)),
            pl.BlockSpec((TBA, LOGIT_PAD), lambda i: (i, 0)),
        ],
        out_shape=[
            jax.ShapeDtypeStruct((B, D), jnp.bfloat16),
            jax.ShapeDtypeStruct((B, LOGIT_PAD), jnp.float32),
            jax.ShapeDtypeStruct((B, LOGIT_PAD), jnp.int32),
        ],
    )(hidden, feats, ln_gamma.reshape(1, D), ln_beta.reshape(1, D),
      rW1[:D], rW1[D:], rb1.reshape(1, RH), rW2p, rb2p)


def _routing_metadata(topi, wts, alpha):
    pair_e = topi.reshape(B * K)
    order = jnp.argsort(pair_e, stable=True)
    counts = jnp.bincount(pair_e, length=E)
    o = jnp.concatenate([jnp.zeros((1,), counts.dtype),
                         jnp.cumsum(counts)[:-1]])
    aligned = ((counts + TS - 1) // TS) * TS
    pstart = jnp.concatenate([jnp.zeros((1,), counts.dtype),
                              jnp.cumsum(aligned)[:-1]])
    s = jnp.arange(NSLOT, dtype=jnp.int32)
    e_of_s = jnp.clip(
        jnp.searchsorted(pstart, s, side="right") - 1, 0, E - 1
    ).astype(jnp.int32)
    local = s - pstart[e_of_s]
    valid = local < counts[e_of_s]
    src = jnp.clip(o[e_of_s] + local, 0, B * K - 1)
    pair = jnp.where(valid, order[src], 0).astype(jnp.int32)
    slot_token = pair // K
    slot_w = wts[slot_token, e_of_s] * valid.astype(jnp.float32) * alpha
    block_expert = e_of_s.reshape(NBLK, TS)[:, 0].astype(jnp.int32)
    block_valid = valid.reshape(NBLK, TS).any(axis=1).astype(jnp.int32)
    e_sorted = pair_e[order]
    j = jnp.arange(B * K, dtype=jnp.int32)
    pos_sorted = pstart[e_sorted] + (j - o[e_sorted])
    pos = jnp.zeros((B * K,), jnp.int32).at[order].set(
        pos_sorted.astype(jnp.int32))
    return slot_token, slot_w, block_expert, block_valid, pos


def _bf16_to_i32(x):
    return jax.lax.bitcast_convert_type(
        x.reshape(*x.shape[:-1], x.shape[-1] // 2, 2), jnp.int32)


def _i32_to_bf16(x):
    return jax.lax.bitcast_convert_type(x, jnp.bfloat16).reshape(
        *x.shape[:-1], x.shape[-1] * 2)


HW32 = D // 2 // 128
EFW32 = E * EFW // 2 // 128


def _sc_gather_x(slot_token, h3d, ef3d):
    rows_per_w = NSLOT // NW
    chunk = 64
    nchunk = rows_per_w // chunk
    mesh = plsc.VectorSubcoreMesh(core_axis_name="c", subcore_axis_name="s")

    @functools.partial(
        pl.kernel,
        out_type=[
            jax.ShapeDtypeStruct((NSLOT, HW32, 128), jnp.int32),
            jax.ShapeDtypeStruct((NSLOT, EFW32, 128), jnp.int32),
        ],
        mesh=mesh,
        scratch_types=[
            pltpu.VMEM((chunk,), jnp.int32),
            pltpu.VMEM((chunk, HW32, 128), jnp.int32),
            pltpu.VMEM((chunk, EFW32, 128), jnp.int32),
            pltpu.SemaphoreType.DMA,
            pltpu.SemaphoreType.DMA,
        ],
    )
    def body(idx_hbm, h_hbm, ef_hbm, outh_hbm, outef_hbm,
             idx_v, h_v, ef_v, sem1, sem2):
        wid = lax.axis_index("s") * 2 + lax.axis_index("c")
        for c in range(nchunk):
            base = wid * rows_per_w + c * chunk
            pltpu.sync_copy(idx_hbm.at[pl.ds(base, chunk)], idx_v)
            cp1 = pltpu.async_copy(h_hbm.at[idx_v], h_v, sem1)
            cp2 = pltpu.async_copy(ef_hbm.at[idx_v], ef_v, sem2)
            cp1.wait()
            cp2.wait()
            pltpu.sync_copy(h_v, outh_hbm.at[pl.ds(base, chunk)])
            pltpu.sync_copy(ef_v, outef_hbm.at[pl.ds(base, chunk)])

    return body(slot_token, h3d, ef3d)


def _sc_gather_pairs(pos, op3d):
    rows_per_w = (B * K) // NW
    chunk = 64
    nchunk = rows_per_w // chunk
    mesh = plsc.VectorSubcoreMesh(core_axis_name="c", subcore_axis_name="s")

    @functools.partial(
        pl.kernel,
        out_type=jax.ShapeDtypeStruct((B * K, HW32, 128), jnp.int32),
        mesh=mesh,
        scratch_types=[
            pltpu.VMEM((chunk,), jnp.int32),
            pltpu.VMEM((chunk, HW32, 128), jnp.int32),
            pltpu.SemaphoreType.DMA,
        ],
    )
    def body(pos_hbm, op_hbm, out_hbm, idx_v, row_v, sem):
        wid = lax.axis_index("s") * 2 + lax.axis_index("c")
        for c in range(nchunk):
            base = wid * rows_per_w + c * chunk
            pltpu.sync_copy(pos_hbm.at[pl.ds(base, chunk)], idx_v)
            pltpu.async_copy(op_hbm.at[idx_v], row_v, sem).wait()
            pltpu.sync_copy(row_v, out_hbm.at[pl.ds(base, chunk)])

    return body(pos, op3d)


def _expert_kernel(be_ref, bv_ref, xh_ref, xef_ref, w1a_ref, w1b_ref,
                   b1_ref, w2_ref, b2_ref, w3_ref, b3_ref, ws_ref, out_ref):
    b = pl.program_id(0)

    @pl.when(bv_ref[b] != 0)
    def _():
        x1 = jnp.dot(xh_ref[...], w1a_ref[0].astype(jnp.bfloat16),
                     preferred_element_type=jnp.float32)
        x1 = x1 + jnp.dot(xef_ref[...], w1b_ref[0].astype(jnp.bfloat16),
                          preferred_element_type=jnp.float32)
        h1 = _gelu_exact(x1 + b1_ref[0])
        h2 = jnp.dot(h1.astype(jnp.bfloat16), w2_ref[0].astype(jnp.bfloat16),
                     preferred_element_type=jnp.float32)
        h2 = _gelu_exact(h2 + b2_ref[0])
        oe = jnp.dot(h2.astype(jnp.bfloat16), w3_ref[0].astype(jnp.bfloat16),
                     preferred_element_type=jnp.float32)
        oe = oe + b3_ref[0]
        out_ref[...] = (oe * ws_ref[0]).astype(jnp.bfloat16)


def _run_experts(block_expert, block_valid, xh, xef, We1, We1bp,
                 be1r, We2, be2r, We3, be3r, wslots):
    grid_spec = pltpu.PrefetchScalarGridSpec(
        num_scalar_prefetch=2,
        grid=(NBLK,),
        in_specs=[
            pl.BlockSpec((TS, D), lambda b, be, bv: (b, 0)),
            pl.BlockSpec((TS, E * EFW), lambda b, be, bv: (b, 0)),
            pl.BlockSpec((1, D, H), lambda b, be, bv: (be[b], 0, 0)),
            pl.BlockSpec((1, E * EFW, H), lambda b, be, bv: (be[b], 0, 0)),
            pl.BlockSpec((1, 1, H), lambda b, be, bv: (be[b], 0, 0)),
            pl.BlockSpec((1, H, H), lambda b, be, bv: (be[b], 0, 0)),
            pl.BlockSpec((1, 1, H), lambda b, be, bv: (be[b], 0, 0)),
            pl.BlockSpec((1, H, D), lambda b, be, bv: (be[b], 0, 0)),
            pl.BlockSpec((1, 1, D), lambda b, be, bv: (be[b], 0, 0)),
            pl.BlockSpec((1, TS, 1), lambda b, be, bv: (b, 0, 0)),
        ],
        out_specs=pl.BlockSpec((TS, D), lambda b, be, bv: (b, 0)),
    )
    return pl.pallas_call(
        _expert_kernel,
        grid_spec=grid_spec,
        out_shape=jax.ShapeDtypeStruct((NSLOT, D), jnp.bfloat16),
    )(block_expert, block_valid, xh, xef, We1, We1bp, be1r, We2, be2r,
      We3, be3r, wslots)


def _combine_kernel(hid_ref, op_ref, out_ref):
    op = op_ref[...].astype(jnp.float32)
    out_ref[...] = hid_ref[...] + op[:, :D] + op[:, D:]


def _run_combine(hidden, op_pairs2d):
    return pl.pallas_call(
        _combine_kernel,
        grid=(B // TBA,),
        in_specs=[
            pl.BlockSpec((TBA, D), lambda i: (i, 0)),
            pl.BlockSpec((TBA, K * D), lambda i: (i, 0)),
        ],
        out_specs=pl.BlockSpec((TBA, D), lambda i: (i, 0)),
        out_shape=jax.ShapeDtypeStruct((B, D), jnp.float32),
    )(hidden, op_pairs2d)


def kernel(hidden, feature_bank, expert_bank_idx, ln_gamma, ln_beta,
           rW1, rb1, rW2, rb2, We1, be1, We2, be2, We3, be3, alpha):
    feats = feature_bank.reshape(B, NC * FB)
    rW2p = jnp.zeros((RH, LOGIT_PAD), jnp.float32).at[:, :E].set(rW2)
    rb2p = jnp.full((1, LOGIT_PAD), -1e30, jnp.float32).at[0, :E].set(rb2)
    h_bf, wts, idx_out = _run_router(hidden, feats, ln_gamma, ln_beta,
                                     rW1, rb1, rW2p, rb2p)
    topi = idx_out[:, :K]

    slot_token, slot_w, block_expert, block_valid, pos = _routing_metadata(
        topi, wts[:, :E], alpha)

    ef_all = jnp.take(feature_bank, expert_bank_idx.reshape(-1), axis=1)
    ef_all = ef_all.reshape(B, E * EFW).astype(jnp.bfloat16)

    xh3, xef3 = _sc_gather_x(slot_token,
                             _bf16_to_i32(h_bf).reshape(B, HW32, 128),
                             _bf16_to_i32(ef_all).reshape(B, EFW32, 128))

    We1bp = jnp.zeros((E, E * EFW, H), jnp.float32)
    er = jnp.arange(E)
    We1bp = We1bp.at[er[:, None], EFW * er[:, None] + jnp.arange(EFW)[None, :]
                     ].set(We1[:, D:, :])

    xh = _i32_to_bf16(xh3.reshape(NSLOT, HW32 * 128))
    xef = _i32_to_bf16(xef3.reshape(NSLOT, EFW32 * 128))
    op = _run_experts(block_expert, block_valid, xh, xef,
                      We1, We1bp, be1.reshape(E, 1, H), We2,
                      be2.reshape(E, 1, H), We3, be3.reshape(E, 1, D),
                      slot_w.reshape(NBLK, TS, 1))

    op_pairs = _sc_gather_pairs(pos, _bf16_to_i32(op).reshape(
        NSLOT, HW32, 128))
    return _run_combine(hidden,
                        _i32_to_bf16(op_pairs.reshape(B, K * HW32 * 128)))
